# baseline (speedup 1.0000x reference)
import functools

import jax
import jax.numpy as jnp
from jax import lax
from jax.experimental import pallas as pl
from jax.experimental.pallas import tpu as pltpu
from jax.experimental.pallas import tpu_sc as plsc

B = 16384
D = 16
L = 16
NC = 2
NS = 16
NW = NC * NS
BPW = B // NW

_mesh = plsc.VectorSubcoreMesh(core_axis_name="c", subcore_axis_name="s")


@functools.partial(
    pl.kernel,
    mesh=_mesh,
    compiler_params=pltpu.CompilerParams(
        needs_layout_passes=False, use_tc_tiling_on_sc=False),
    out_type=jax.ShapeDtypeStruct((B,), jnp.float32),
    scratch_types=[
        pltpu.VMEM((BPW,), jnp.int32),
        pltpu.VMEM((BPW,), jnp.int32),
        pltpu.VMEM((BPW, D), jnp.float32),
        pltpu.VMEM((BPW, D), jnp.float32),
        pltpu.VMEM((BPW,), jnp.float32),
        pltpu.SemaphoreType.DMA,
        pltpu.SemaphoreType.DMA,
    ],
)
def _sc_dot(user_hbm, item_hbm, ut_hbm, it_hbm, out_hbm,
            uidx_v, iidx_v, urows_v, irows_v, out_v, usem, isem):
    wid = lax.axis_index("s") * NC + lax.axis_index("c")
    base = wid * BPW

    pltpu.sync_copy(user_hbm.at[pl.ds(base, BPW)], uidx_v)
    pltpu.sync_copy(item_hbm.at[pl.ds(base, BPW)], iidx_v)
    ucp = pltpu.async_copy(ut_hbm.at[uidx_v], urows_v, usem)
    icp = pltpu.async_copy(it_hbm.at[iidx_v], irows_v, isem)
    ucp.wait()
    icp.wait()

    lane = lax.iota(jnp.int32, L)

    def blk_body(blk, carry):
        res = jnp.zeros((L,), jnp.float32)
        for j in range(L):
            row = blk * L + j
            p = urows_v[row] * irows_v[row]
            s = jnp.sum(p)
            res = jnp.where(lane == j, s, res)
        out_v[pl.ds(blk * L, L)] = 1.0 / (1.0 + jnp.exp(-res))
        return carry

    lax.fori_loop(0, BPW // L, blk_body, 0)

    pltpu.sync_copy(out_v, out_hbm.at[pl.ds(base, BPW)])


def kernel(user, item, user_table, item_table):
    return _sc_dot(user.astype(jnp.int32), item.astype(jnp.int32),
                   user_table, item_table)


# speedup vs baseline: 5.3008x; 5.3008x over previous
import jax
import jax.numpy as jnp
from jax import lax
from jax.experimental import pallas as pl
from jax.experimental.pallas import tpu as pltpu
from jax.experimental.pallas import tpu_sc as plsc

B = 16384
D = 16
L = 16
NC = 2
NS = 16
NW = NC * NS
BPW = B // NW
NB = 4
W = 128

_mesh = plsc.VectorSubcoreMesh(core_axis_name="c", subcore_axis_name="s")

_SCRATCH = [
    pltpu.VMEM((BPW,), jnp.int32),
    pltpu.VMEM((BPW,), jnp.int32),
    pltpu.VMEM((NB, D, W), jnp.float32),
    pltpu.VMEM((NB, D, W), jnp.float32),
    pltpu.VMEM((BPW,), jnp.float32),
] + [pltpu.SemaphoreType.DMA] * (2 * NB)


def _sc_dot_body(user_hbm, item_hbm, vtu_hbm, vti_hbm, out_hbm,
                 uidx_v, iidx_v, ubuf, ibuf, out_v, *sems):
    usems = sems[:NB]
    isems = sems[NB:]
    wid = lax.axis_index("s") * NC + lax.axis_index("c")
    base = wid * BPW

    pltpu.sync_copy(user_hbm.at[pl.ds(base, BPW)], uidx_v)
    pltpu.sync_copy(item_hbm.at[pl.ds(base, BPW)], iidx_v)

    lane = lax.iota(jnp.int32, L)

    def issue(wu, wi, slot):
        wu = pl.multiple_of(wu, W)
        wi = pl.multiple_of(wi, W)
        pltpu.async_copy(vtu_hbm.at[:, pl.ds(wu, W)], ubuf.at[slot],
                         usems[slot])
        pltpu.async_copy(vti_hbm.at[:, pl.ds(wi, W)], ibuf.at[slot],
                         isems[slot])

    def wait(slot):
        pltpu.make_async_copy(
            vtu_hbm.at[:, pl.ds(0, W)], ubuf.at[slot], usems[slot]).wait()
        pltpu.make_async_copy(
            vti_hbm.at[:, pl.ds(0, W)], ibuf.at[slot], isems[slot]).wait()

    u0 = uidx_v[pl.ds(0, L)]
    i0 = iidx_v[pl.ds(0, L)]
    uw0 = (u0 >> 7) << 7
    iw0 = (i0 >> 7) << 7
    for b in range(NB):
        issue(uw0[b], iw0[b], b)

    def blk_body(blk, carry):
        u16 = uidx_v[pl.ds(blk * L, L)]
        i16 = iidx_v[pl.ds(blk * L, L)]
        nxt = jnp.minimum((blk + 1) * L, BPW - L)
        u16n = uidx_v[pl.ds(nxt, L)]
        i16n = iidx_v[pl.ds(nxt, L)]
        uw = (u16 >> 7) << 7
        iw = (i16 >> 7) << 7
        uwn = (u16n >> 7) << 7
        iwn = (i16n >> 7) << 7
        ulo = u16 & 127
        ilo = i16 & 127
        res = jnp.zeros((L,), jnp.float32)
        for t in range(L):
            slot = t % NB
            wait(slot)
            uvec = jnp.broadcast_to(ulo[t], (L,))
            ivec = jnp.broadcast_to(ilo[t], (L,))
            urow = plsc.load_gather(ubuf.at[slot], [lane, uvec])
            irow = plsc.load_gather(ibuf.at[slot], [lane, ivec])
            s = jnp.sum(urow * irow)
            res = jnp.where(lane == t, s, res)
            if t + NB < L:
                issue(uw[t + NB], iw[t + NB], slot)
            else:
                issue(uwn[t + NB - L], iwn[t + NB - L], slot)
        out_v[pl.ds(blk * L, L)] = 1.0 / (1.0 + jnp.exp(-res))
        return carry

    lax.fori_loop(0, BPW // L, blk_body, 0)

    for b in range(NB):
        wait(b)

    pltpu.sync_copy(out_v, out_hbm.at[pl.ds(base, BPW)])


_sc_dot = pl.kernel(
    _sc_dot_body,
    mesh=_mesh,
    compiler_params=pltpu.CompilerParams(needs_layout_passes=False),
    out_type=jax.ShapeDtypeStruct((B,), jnp.float32),
    scratch_types=_SCRATCH,
)


def kernel(user, item, user_table, item_table):
    vtu = user_table.T
    vti = item_table.T
    return _sc_dot(user.astype(jnp.int32), item.astype(jnp.int32), vtu, vti)


# speedup vs baseline: 5.7590x; 1.0864x over previous
import jax
import jax.numpy as jnp
from jax import lax
from jax.experimental import pallas as pl
from jax.experimental.pallas import tpu as pltpu
from jax.experimental.pallas import tpu_sc as plsc

B = 16384
D = 16
L = 16
NC = 2
NS = 16
NW = NC * NS
BPW = B // NW
G = 4
NSLOT = 4
NG = BPW // G
W = 128
WSH = 7
WM = W - 1

_mesh = plsc.VectorSubcoreMesh(core_axis_name="c", subcore_axis_name="s")

_SCRATCH = [
    pltpu.VMEM((BPW + L,), jnp.int32),
    pltpu.VMEM((BPW + L,), jnp.int32),
    pltpu.VMEM((NSLOT, G, D, W), jnp.float32),
    pltpu.VMEM((NSLOT, G, D, W), jnp.float32),
    pltpu.VMEM((BPW,), jnp.float32),
] + [pltpu.SemaphoreType.DMA] * (2 * NSLOT)


def _sc_dot_body(user_hbm, item_hbm, vtu_hbm, vti_hbm, out_hbm,
                 uidx_v, iidx_v, ubuf, ibuf, out_v, *sems):
    usems = sems[:NSLOT]
    isems = sems[NSLOT:]
    wid = lax.axis_index("s") * NC + lax.axis_index("c")
    base = wid * BPW

    pltpu.sync_copy(user_hbm.at[pl.ds(base, BPW)], uidx_v.at[pl.ds(0, BPW)])
    pltpu.sync_copy(item_hbm.at[pl.ds(base, BPW)], iidx_v.at[pl.ds(0, BPW)])

    lane = lax.iota(jnp.int32, L)

    def issue_group(g, slot):
        g = jnp.minimum(g, NG - 1)
        ug = uidx_v[pl.ds(g * G, L)]
        ig = iidx_v[pl.ds(g * G, L)]
        uw = (ug >> WSH) << WSH
        iw = (ig >> WSH) << WSH
        for k in range(G):
            wu = pl.multiple_of(uw[k], W)
            wi = pl.multiple_of(iw[k], W)
            pltpu.async_copy(vtu_hbm.at[:, pl.ds(wu, W)], ubuf.at[slot, k],
                             usems[slot])
            pltpu.async_copy(vti_hbm.at[:, pl.ds(wi, W)], ibuf.at[slot, k],
                             isems[slot])

    def wait_group(slot):
        for k in range(G):
            pltpu.make_async_copy(vtu_hbm.at[:, pl.ds(0, W)],
                                  ubuf.at[slot, k], usems[slot]).wait()
            pltpu.make_async_copy(vti_hbm.at[:, pl.ds(0, W)],
                                  ibuf.at[slot, k], isems[slot]).wait()

    issue_group(0, 0)
    issue_group(1, 1)

    def blk_body(m, carry):
        res = jnp.zeros((L,), jnp.float32)
        u16 = uidx_v[pl.ds(m * L, L)]
        i16 = iidx_v[pl.ds(m * L, L)]
        ulo = u16 & WM
        ilo = i16 & WM
        for b in range(NSLOT):
            g = m * NSLOT + b
            wait_group(b)
            issue_group(g + 2, (b + 2) % NSLOT)
            for k in range(G):
                t = b * G + k
                uvec = jnp.broadcast_to(ulo[t], (L,))
                ivec = jnp.broadcast_to(ilo[t], (L,))
                urow = plsc.load_gather(ubuf.at[b, k], [lane, uvec])
                irow = plsc.load_gather(ibuf.at[b, k], [lane, ivec])
                s = jnp.sum(urow * irow)
                res = jnp.where(lane == t, s, res)
        out_v[pl.ds(m * L, L)] = 1.0 / (1.0 + jnp.exp(-res))
        return carry

    lax.fori_loop(0, NG // NSLOT, blk_body, 0)

    wait_group(0)
    wait_group(1)

    pltpu.sync_copy(out_v, out_hbm.at[pl.ds(base, BPW)])


_sc_dot = pl.kernel(
    _sc_dot_body,
    mesh=_mesh,
    compiler_params=pltpu.CompilerParams(needs_layout_passes=False),
    out_type=jax.ShapeDtypeStruct((B,), jnp.float32),
    scratch_types=_SCRATCH,
)


def kernel(user, item, user_table, item_table):
    vtu = user_table.T
    vti = item_table.T
    return _sc_dot(user.astype(jnp.int32), item.astype(jnp.int32), vtu, vti)


# speedup vs baseline: 6.4441x; 1.1190x over previous
import jax
import jax.numpy as jnp
from jax import lax
from jax.experimental import pallas as pl
from jax.experimental.pallas import tpu as pltpu
from jax.experimental.pallas import tpu_sc as plsc

B = 16384
D = 16
L = 16
NC = 2
NS = 16
NW = NC * NS
BPW = B // NW
G = 4
NSLOT = 4
NG = BPW // G
W = 128
WSH = 7
WM = W - 1

_mesh = plsc.VectorSubcoreMesh(core_axis_name="c", subcore_axis_name="s")

_SCRATCH = [
    pltpu.VMEM((BPW + L,), jnp.int32),
    pltpu.VMEM((BPW + L,), jnp.int32),
    pltpu.VMEM((NSLOT, G, D, W), jnp.float32),
    pltpu.VMEM((NSLOT, G, D, W), jnp.float32),
    pltpu.VMEM((BPW,), jnp.float32),
] + [pltpu.SemaphoreType.DMA] * (2 * NSLOT)


def _sc_dot_body(user_hbm, item_hbm, vtu_hbm, vti_hbm, out_hbm,
                 uidx_v, iidx_v, ubuf, ibuf, out_v, *sems):
    usems = sems[:NSLOT]
    isems = sems[NSLOT:]
    wid = lax.axis_index("s") * NC + lax.axis_index("c")
    base = wid * BPW

    pltpu.sync_copy(user_hbm.at[pl.ds(base, BPW)], uidx_v.at[pl.ds(0, BPW)])
    pltpu.sync_copy(item_hbm.at[pl.ds(base, BPW)], iidx_v.at[pl.ds(0, BPW)])

    lane = lax.iota(jnp.int32, L)

    def issue_group(g, slot):
        g = jnp.minimum(g, NG - 1)
        ug = uidx_v[pl.ds(g * G, L)]
        ig = iidx_v[pl.ds(g * G, L)]
        uw = (ug >> WSH) << WSH
        iw = (ig >> WSH) << WSH
        for k in range(G):
            wu = pl.multiple_of(uw[k], W)
            wi = pl.multiple_of(iw[k], W)
            pltpu.async_copy(vtu_hbm.at[:, pl.ds(wu, W)], ubuf.at[slot, k],
                             usems[slot])
            pltpu.async_copy(vti_hbm.at[:, pl.ds(wi, W)], ibuf.at[slot, k],
                             isems[slot])

    def wait_group(slot):
        for k in range(G):
            pltpu.make_async_copy(vtu_hbm.at[:, pl.ds(0, W)],
                                  ubuf.at[slot, k], usems[slot]).wait()
            pltpu.make_async_copy(vti_hbm.at[:, pl.ds(0, W)],
                                  ibuf.at[slot, k], isems[slot]).wait()

    issue_group(0, 0)
    issue_group(1, 1)
    issue_group(2, 2)

    def blk_body(m, carry):
        res = jnp.zeros((L,), jnp.float32)
        u16 = uidx_v[pl.ds(m * L, L)]
        i16 = iidx_v[pl.ds(m * L, L)]
        ulo = u16 & WM
        ilo = i16 & WM
        for b in range(NSLOT):
            g = m * NSLOT + b
            wait_group(b)
            issue_group(g + 3, (b + 3) % NSLOT)
            for k in range(G):
                t = b * G + k
                uvec = jnp.broadcast_to(ulo[t], (L,))
                ivec = jnp.broadcast_to(ilo[t], (L,))
                urow = plsc.load_gather(ubuf.at[b, k], [lane, uvec])
                irow = plsc.load_gather(ibuf.at[b, k], [lane, ivec])
                s = jnp.sum(urow * irow)
                res = jnp.where(lane == t, s, res)
        out_v[pl.ds(m * L, L)] = 1.0 / (1.0 + jnp.exp(-res))
        return carry

    lax.fori_loop(0, NG // NSLOT, blk_body, 0)

    wait_group(0)
    wait_group(1)
    wait_group(2)

    pltpu.sync_copy(out_v, out_hbm.at[pl.ds(base, BPW)])


_sc_dot = pl.kernel(
    _sc_dot_body,
    mesh=_mesh,
    compiler_params=pltpu.CompilerParams(needs_layout_passes=False),
    out_type=jax.ShapeDtypeStruct((B,), jnp.float32),
    scratch_types=_SCRATCH,
)


def kernel(user, item, user_table, item_table):
    vtu = user_table.T
    vti = item_table.T
    return _sc_dot(user.astype(jnp.int32), item.astype(jnp.int32), vtu, vti)


# speedup vs baseline: 6.4767x; 1.0051x over previous
import jax
import jax.numpy as jnp
from jax import lax
from jax.experimental import pallas as pl
from jax.experimental.pallas import tpu as pltpu
from jax.experimental.pallas import tpu_sc as plsc

B = 16384
D = 16
L = 16
NC = 2
NS = 16
NW = NC * NS
BPW = B // NW
G = 4
NSLOT = 4
NG = BPW // G
W = 128
WSH = 7
WM = W - 1

_mesh = plsc.VectorSubcoreMesh(core_axis_name="c", subcore_axis_name="s")

_SCRATCH = [
    pltpu.VMEM((BPW + L,), jnp.int32),
    pltpu.VMEM((BPW + L,), jnp.int32),
    pltpu.VMEM((NSLOT, G, D, W), jnp.float32),
    pltpu.VMEM((NSLOT, G, D, W), jnp.float32),
    pltpu.VMEM((BPW,), jnp.float32),
] + [pltpu.SemaphoreType.DMA] * (2 * NSLOT)


def _sc_dot_body(user_hbm, item_hbm, vtu_hbm, vti_hbm, out_hbm,
                 uidx_v, iidx_v, ubuf, ibuf, out_v, *sems):
    usems = sems[:NSLOT]
    isems = sems[NSLOT:]
    wid = lax.axis_index("s") * NC + lax.axis_index("c")
    base = wid * BPW

    pltpu.sync_copy(user_hbm.at[pl.ds(base, BPW)], uidx_v.at[pl.ds(0, BPW)])
    pltpu.sync_copy(item_hbm.at[pl.ds(base, BPW)], iidx_v.at[pl.ds(0, BPW)])

    lane = lax.iota(jnp.int32, L)

    def issue_group(g, slot):
        g = jnp.minimum(g, NG - 1)
        ug = uidx_v[pl.ds(g * G, L)]
        ig = iidx_v[pl.ds(g * G, L)]
        uw = (ug >> WSH) << WSH
        iw = (ig >> WSH) << WSH
        for k in range(G):
            wu = pl.multiple_of(uw[k], W)
            wi = pl.multiple_of(iw[k], W)
            for h in range(2):
                hs = pl.ds(h * 8, 8)
                pltpu.async_copy(vtu_hbm.at[hs, pl.ds(wu, W)],
                                 ubuf.at[slot, k, hs], usems[slot])
                pltpu.async_copy(vti_hbm.at[hs, pl.ds(wi, W)],
                                 ibuf.at[slot, k, hs], isems[slot])

    def wait_group(slot):
        for k in range(G):
            pltpu.make_async_copy(vtu_hbm.at[:, pl.ds(0, W)],
                                  ubuf.at[slot, k], usems[slot]).wait()
            pltpu.make_async_copy(vti_hbm.at[:, pl.ds(0, W)],
                                  ibuf.at[slot, k], isems[slot]).wait()

    issue_group(0, 0)
    issue_group(1, 1)
    issue_group(2, 2)

    def blk_body(m, carry):
        res = jnp.zeros((L,), jnp.float32)
        u16 = uidx_v[pl.ds(m * L, L)]
        i16 = iidx_v[pl.ds(m * L, L)]
        ulo = u16 & WM
        ilo = i16 & WM
        for b in range(NSLOT):
            g = m * NSLOT + b
            wait_group(b)
            issue_group(g + 3, (b + 3) % NSLOT)
            for k in range(G):
                t = b * G + k
                uvec = jnp.broadcast_to(ulo[t], (L,))
                ivec = jnp.broadcast_to(ilo[t], (L,))
                urow = plsc.load_gather(ubuf.at[b, k], [lane, uvec])
                irow = plsc.load_gather(ibuf.at[b, k], [lane, ivec])
                s = jnp.sum(urow * irow)
                res = jnp.where(lane == t, s, res)
        out_v[pl.ds(m * L, L)] = 1.0 / (1.0 + jnp.exp(-res))
        return carry

    lax.fori_loop(0, NG // NSLOT, blk_body, 0)

    wait_group(0)
    wait_group(1)
    wait_group(2)

    pltpu.sync_copy(out_v, out_hbm.at[pl.ds(base, BPW)])


_sc_dot = pl.kernel(
    _sc_dot_body,
    mesh=_mesh,
    compiler_params=pltpu.CompilerParams(needs_layout_passes=False),
    out_type=jax.ShapeDtypeStruct((B,), jnp.float32),
    scratch_types=_SCRATCH,
)


def kernel(user, item, user_table, item_table):
    vtu = user_table.T
    vti = item_table.T
    return _sc_dot(user.astype(jnp.int32), item.astype(jnp.int32), vtu, vti)
